# baseline (speedup 1.0000x reference)
import jax
import jax.numpy as jnp
from jax.experimental import pallas as pl

NGRID_LL = 721 * 1440
NGRID_CS = 6 * 64 * 64
B, T, C = 1, 2, 7


def _spmm_t(xmat, rows, cols, vals, n_rows):
    contrib = jnp.take(xmat, cols, axis=1) * vals
    return jax.ops.segment_sum(contrib.T, rows, num_segments=n_rows).T


def _affine_kernel(ll_ref, scale_ref, center_ref, out_ref):
    out_ref[...] = ll_ref[...] * scale_ref[...] + center_ref[...]


def kernel(x, landsea_mask, topographic_height, latgrid, longrid, center, scale, Min_vals, Mout_vals, W, Min_rows, Min_cols, Mout_rows, Mout_cols):
    xn = (x - center) / scale
    xm = xn.reshape(B * T * C, NGRID_LL)
    cs = _spmm_t(xm, Min_rows, Min_cols, Min_vals, NGRID_CS).reshape(B, T, C, 6, 64, 64)
    d2r = jnp.pi / 180.0
    steps = []
    for i in range(T):
        tisr = jnp.maximum(jnp.cos(latgrid * d2r) * jnp.cos(longrid * d2r + 0.5 * i), 0.0) - 1.0 / jnp.pi
        tisr = jnp.broadcast_to(tisr[None, None], (B, 1, 6, 64, 64))
        steps.append(jnp.concatenate([cs[:, i], tisr], axis=1))
    inp = jnp.concatenate(steps, axis=1)
    lsm = jnp.broadcast_to(landsea_mask[None, None], (B, 1, 6, 64, 64))
    topo = jnp.broadcast_to(((topographic_height - 3724.0) / 8349.0)[None, None], (B, 1, 6, 64, 64))
    inp = jnp.concatenate([inp, lsm, topo], axis=1)
    y = jnp.einsum('oc,bcfhw->bofhw', W, inp)
    out = jnp.stack(jnp.split(y, 2, axis=1), axis=1)
    om = out.reshape(B * T * C, NGRID_CS)
    ll = _spmm_t(om, Mout_rows, Mout_cols, Mout_vals, NGRID_LL).reshape(B * T * C, 1, NGRID_LL)
    scale_b = jnp.tile(jnp.broadcast_to(scale.reshape(C, 1), (C, NGRID_LL)), (T, 1)).reshape(T * C, 1, NGRID_LL)
    center_b = jnp.tile(jnp.broadcast_to(center.reshape(C, 1), (C, NGRID_LL)), (T, 1)).reshape(T * C, 1, NGRID_LL)
    res = pl.pallas_call(
        _affine_kernel,
        out_shape=jax.ShapeDtypeStruct((B * T * C, 1, NGRID_LL), jnp.float32),
        grid=(14,),
        in_specs=[pl.BlockSpec((1, 1, NGRID_LL), lambda i: (i, 0, 0))] * 3,
        out_specs=pl.BlockSpec((1, 1, NGRID_LL), lambda i: (i, 0, 0)),
    )(ll, scale_b, center_b)
    return res.reshape(B, T, C, 721, 1440)


# speedup vs baseline: 5.4194x; 5.4194x over previous
import functools
import math

import jax
import jax.numpy as jnp
from jax import lax
from jax.experimental import pallas as pl
from jax.experimental.pallas import tpu as pltpu
from jax.experimental.pallas import tpu_sc as plsc

N_LL = 721 * 1440
N_CS = 6 * 64 * 64
NNZ_IN = 221184
NNZ_OUT = 4152960
ROWS_IN_PAD = 16 * 14 * 8
NNZ_IN_PAD = ROWS_IN_PAD * 128
ROWS_OUT_PAD = 16 * 127 * 16
NNZ_OUT_PAD = ROWS_OUT_PAD * 128
LL_TS = 64800
LL_TAIL = N_LL - 16 * LL_TS
LL_PIECE = 7200
CS_PER_TILE = N_CS // 16
D2R = math.pi / 180.0

@functools.cache
def _mesh():
    return plsc.VectorSubcoreMesh(core_axis_name="c", subcore_axis_name="s",
                                  num_cores=2, num_subcores=16)


def _in_remap_body(x_ref, minr_ref, minc_ref, minv_ref, cs_ref,
                   rows_v, cols_v, vals_v, idx_v, gbuf, contrib, zbuf, abuf,
                   acc, gsem, ssem):
    c = lax.axis_index("c")
    s = lax.axis_index("s")
    zv = jnp.zeros((16,), jnp.float32)
    def zfill(j, _):
        zbuf[pl.ds(j * 16, 16)] = zv
        return 0
    lax.fori_loop(0, CS_PER_TILE // 16, zfill, 0)

    def chunk(k, ch, gather):
        r0 = (s * 14 + k) * 8
        pltpu.sync_copy(minr_ref.at[pl.ds(r0, 8)], rows_v)
        pltpu.sync_copy(minc_ref.at[pl.ds(r0, 8)], cols_v)
        pltpu.sync_copy(minv_ref.at[pl.ds(r0, 8)], vals_v)
        if gather:
            base = ch * N_LL
            for j in range(8):
                for g in range(8):
                    idx_v[j, pl.ds(g * 16, 16)] = cols_v[j, pl.ds(g * 16, 16)] + base
            descs = [pltpu.async_copy(x_ref.at[idx_v.at[j]], gbuf.at[j], gsem)
                     for j in range(8)]
            for d in descs:
                d.wait()
            for j in range(8):
                for g in range(8):
                    sl = pl.ds(g * 16, 16)
                    contrib[j, sl] = gbuf[j, sl] * vals_v[j, sl]
            src = contrib
        else:
            src = vals_v
        sdescs = [pltpu.async_copy(src.at[j], acc.at[rows_v.at[j]], ssem, add=True)
                  for j in range(8)]
        for d in sdescs:
            d.wait()

    def channel_pass(ch, gather):
        pltpu.sync_copy(zbuf, acc.at[pl.ds(s * CS_PER_TILE, CS_PER_TILE)])
        plsc.subcore_barrier()
        def body(k, _):
            chunk(k, ch, gather)
            return 0
        lax.fori_loop(0, 14, body, 0)
        plsc.subcore_barrier()
        pltpu.sync_copy(acc.at[pl.ds(s * CS_PER_TILE, CS_PER_TILE)], abuf)
        pltpu.sync_copy(abuf,
                        cs_ref.at[pl.ds(ch * N_CS + s * CS_PER_TILE, CS_PER_TILE)])

    def ch_loop(i, _):
        channel_pass(2 * i + c, True)
        return 0
    lax.fori_loop(0, 7, ch_loop, 0)

    @pl.when(c == 0)
    def _():
        channel_pass(jnp.int32(14), False)


def _mix_body(cs_ref, lat_ref, lon_ref, lsm_ref, topo_ref, w_ref, cen_ref,
              sc_ref, om_ref):
    cs = cs_ref[...]
    rowsum = cs[14:15]
    csn = (cs[0:14] - cen_ref[...] * rowsum) / sc_ref[...]
    lat = lat_ref[...] * D2R
    lon = lon_ref[...] * D2R
    clat = jnp.cos(lat)
    t0 = jnp.maximum(clat * jnp.cos(lon), 0.0) - 1.0 / math.pi
    t1 = jnp.maximum(clat * jnp.cos(lon + 0.5), 0.0) - 1.0 / math.pi
    topon = (topo_ref[...] - 3724.0) / 8349.0
    w = w_ref[...]
    dot = functools.partial(lax.dot_general,
                            dimension_numbers=(((1,), (0,)), ((), ())),
                            precision=lax.Precision.HIGHEST,
                            preferred_element_type=jnp.float32)
    y = (dot(w[:, 0:7], csn[0:7]) + dot(w[:, 8:15], csn[7:14])
         + w[:, 7:8] * t0 + w[:, 15:16] * t1
         + w[:, 16:17] * lsm_ref[...] + w[:, 17:18] * topon)
    om_ref[...] = y * sc_ref[...]


def _out_remap_body(om_ref, moutr_ref, moutc_ref, moutv_ref, c16_ref, out_ref,
                    om_vmem, rows_v, cols_v, vals_v, contrib, fillb, tailb,
                    cen_v, acc, ssem):
    c = lax.axis_index("c")
    s = lax.axis_index("s")

    def ch_pass(i, _):
        ch = 2 * i + c
        pltpu.sync_copy(om_ref.at[pl.ds(ch * N_CS, N_CS)], om_vmem)
        pltpu.sync_copy(c16_ref.at[pl.ds(ch * 16, 16)], cen_v)
        cvec = cen_v[...]
        def cfill(j, _):
            fillb[pl.ds(j * 16, 16)] = cvec
            return 0
        lax.fori_loop(0, LL_PIECE // 16, cfill, 0)
        def ifill(p, _):
            pltpu.sync_copy(fillb, acc.at[pl.ds(s * LL_TS + p * LL_PIECE, LL_PIECE)])
            return 0
        lax.fori_loop(0, LL_TS // LL_PIECE, ifill, 0)
        @pl.when(s == 0)
        def _():
            pltpu.sync_copy(fillb.at[pl.ds(0, LL_TAIL)],
                            acc.at[pl.ds(16 * LL_TS, LL_TAIL)])
        plsc.subcore_barrier()

        def chunk(k, _):
            r0 = (s * 127 + k) * 16
            pltpu.sync_copy(moutr_ref.at[pl.ds(r0, 16)], rows_v)
            pltpu.sync_copy(moutc_ref.at[pl.ds(r0, 16)], cols_v)
            pltpu.sync_copy(moutv_ref.at[pl.ds(r0, 16)], vals_v)
            for j in range(16):
                for g in range(8):
                    sl = pl.ds(g * 16, 16)
                    gat = plsc.load_gather(om_vmem, [cols_v[j, sl]])
                    contrib[j, sl] = gat * vals_v[j, sl]
            sdescs = [pltpu.async_copy(contrib.at[j], acc.at[rows_v.at[j]],
                                       ssem, add=True) for j in range(16)]
            for d in sdescs:
                d.wait()
            return 0
        lax.fori_loop(0, 127, chunk, 0)
        plsc.subcore_barrier()
        def opiece(p, _):
            o = s * LL_TS + p * LL_PIECE
            pltpu.sync_copy(acc.at[pl.ds(o, LL_PIECE)], fillb)
            pltpu.sync_copy(fillb, out_ref.at[pl.ds(ch * N_LL + o, LL_PIECE)])
            return 0
        lax.fori_loop(0, LL_TS // LL_PIECE, opiece, 0)
        @pl.when(s == 0)
        def _():
            pltpu.sync_copy(acc.at[pl.ds(16 * LL_TS, LL_TAIL)],
                            tailb)
            pltpu.sync_copy(tailb,
                            out_ref.at[pl.ds(ch * N_LL + 16 * LL_TS, LL_TAIL)])
        return 0
    lax.fori_loop(0, 7, ch_pass, 0)


def kernel(x, landsea_mask, topographic_height, latgrid, longrid, center,
           scale, Min_vals, Mout_vals, W, Min_rows, Min_cols, Mout_rows,
           Mout_cols):
    x_flat = x.reshape(14 * N_LL)
    ipad = NNZ_IN_PAD - NNZ_IN
    minr = jnp.concatenate([Min_rows, jnp.zeros((ipad,), jnp.int32)]).reshape(ROWS_IN_PAD, 128)
    minc = jnp.concatenate([Min_cols, jnp.zeros((ipad,), jnp.int32)]).reshape(ROWS_IN_PAD, 128)
    minv = jnp.concatenate([Min_vals, jnp.zeros((ipad,), jnp.float32)]).reshape(ROWS_IN_PAD, 128)
    npad = NNZ_OUT_PAD - NNZ_OUT
    moutr = jnp.concatenate([Mout_rows, jnp.zeros((npad,), jnp.int32)]).reshape(ROWS_OUT_PAD, 128)
    moutc = jnp.concatenate([Mout_cols, jnp.zeros((npad,), jnp.int32)]).reshape(ROWS_OUT_PAD, 128)
    moutv = jnp.concatenate([Mout_vals, jnp.zeros((npad,), jnp.float32)]).reshape(ROWS_OUT_PAD, 128)
    c7 = center.reshape(7)
    s7 = scale.reshape(7)
    c16 = jnp.repeat(jnp.tile(c7, 2), 16)
    cen_tc = jnp.tile(c7, 2).reshape(14, 1)
    sc_tc = jnp.tile(s7, 2).reshape(14, 1)

    in_remap = pl.kernel(
        _in_remap_body,
        out_type=jax.ShapeDtypeStruct((15 * N_CS,), jnp.float32),
        mesh=_mesh(),
        compiler_params=pltpu.CompilerParams(needs_layout_passes=False),
        scratch_types=[
            pltpu.VMEM((8, 128), jnp.int32),
            pltpu.VMEM((8, 128), jnp.int32),
            pltpu.VMEM((8, 128), jnp.float32),
            pltpu.VMEM((8, 128), jnp.int32),
            pltpu.VMEM((8, 128), jnp.float32),
            pltpu.VMEM((8, 128), jnp.float32),
            pltpu.VMEM((CS_PER_TILE,), jnp.float32),
            pltpu.VMEM((CS_PER_TILE,), jnp.float32),
            pltpu.VMEM_SHARED((N_CS,), jnp.float32),
            pltpu.SemaphoreType.DMA,
            pltpu.SemaphoreType.DMA,
        ],
    )
    cs_raw = in_remap(x_flat, minr, minc, minv).reshape(15, N_CS)

    om = pl.pallas_call(
        _mix_body,
        out_shape=jax.ShapeDtypeStruct((14, N_CS), jnp.float32),
    )(cs_raw, latgrid.reshape(1, N_CS), longrid.reshape(1, N_CS),
      landsea_mask.reshape(1, N_CS), topographic_height.reshape(1, N_CS),
      W, cen_tc, sc_tc)

    out_remap = pl.kernel(
        _out_remap_body,
        out_type=jax.ShapeDtypeStruct((14 * N_LL,), jnp.float32),
        mesh=_mesh(),
        compiler_params=pltpu.CompilerParams(needs_layout_passes=False),
        scratch_types=[
            pltpu.VMEM((N_CS,), jnp.float32),
            pltpu.VMEM((16, 128), jnp.int32),
            pltpu.VMEM((16, 128), jnp.int32),
            pltpu.VMEM((16, 128), jnp.float32),
            pltpu.VMEM((16, 128), jnp.float32),
            pltpu.VMEM((LL_PIECE,), jnp.float32),
            pltpu.VMEM((LL_TAIL,), jnp.float32),
            pltpu.VMEM((16,), jnp.float32),
            pltpu.VMEM_SHARED((N_LL,), jnp.float32),
            pltpu.SemaphoreType.DMA,
        ],
    )
    out_flat = out_remap(om.reshape(14 * N_CS), moutr, moutc, moutv, c16)
    return out_flat.reshape(1, 2, 7, 721, 1440)
